# baseline (speedup 1.0000x reference)
import jax
import jax.numpy as jnp
from jax import lax
from jax.experimental import pallas as pl
from jax.experimental.pallas import tpu as pltpu
from jax.experimental.pallas import tpu_sc as plsc

_NC = 2
_NS = 16
_NW = _NC * _NS
_BATCH = 16384
_HIST = 50
_D = 16
_BPW = _BATCH // _NW
_NBC = _BPW // 128


def _body(idx_hbm, table_hbm, out_hbm, idx_v, sidx0, sidx1, rows0, rows1,
          outv0, outv1, gsem0, gsem1, wsem0, wsem1):
    wid = lax.axis_index("s") * _NC + lax.axis_index("c")
    b0 = wid * _BPW
    bc0 = wid * _NBC
    pltpu.sync_copy(idx_hbm.at[pl.ds(b0 * _HIST, _BPW * _HIST)], idx_v)
    lane = lax.iota(jnp.int32, 16)

    def extract(h, sidx):
        for k in range(_BPW // 16):
            pos = lane * _HIST + (k * 16 * _HIST + h)
            sidx[pl.ds(k * 16, 16)] = plsc.load_gather(idx_v, [pos])

    def start(sidx, rows, sem):
        return pltpu.async_copy(table_hbm.at[sidx], rows, sem)

    def wait(sidx, rows, sem):
        pltpu.make_async_copy(table_hbm.at[sidx], rows, sem).wait()

    def transpose(rows, outv):
        for tr in range(2):
            for bcl in range(_NBC):
                for r in range(8):
                    e = jnp.full((16,), tr * 8 + r, jnp.int32)
                    for bk in range(8):
                        ridx = lane + (bcl * 128 + bk * 16)
                        outv[tr, bcl, r, pl.ds(bk * 16, 16)] = (
                            plsc.load_gather(rows, [ridx, e]))

    def start_write(h, outv, wsem):
        return pltpu.async_copy(
            outv, out_hbm.at[h, :, pl.ds(bc0, _NBC)], wsem)

    def wait_write(h, outv, wsem):
        pltpu.make_async_copy(
            outv, out_hbm.at[h, :, pl.ds(bc0, _NBC)], wsem).wait()

    sidx = (sidx0, sidx1)
    rows = (rows0, rows1)
    outv = (outv0, outv1)
    gsem = (gsem0, gsem1)
    wsem = (wsem0, wsem1)

    def step(h, has_wpending, has_next):
        p = h % 2
        wait(sidx[p], rows[p], gsem[p])
        if has_wpending:
            wait_write(h, outv[p], wsem[p])
        transpose(rows[p], outv[p])
        start_write(h, outv[p], wsem[p])
        if has_next:
            extract(h + 2, sidx[p])
            start(sidx[p], rows[p], gsem[p])

    extract(0, sidx0)
    start(sidx0, rows0, gsem0)
    extract(1, sidx1)
    start(sidx1, rows1, gsem1)
    step(0, False, True)
    step(1, False, True)

    def group(g, carry):
        h0 = g * 2

        def half(h, parity):
            pp = (sidx[parity], rows[parity], outv[parity],
                  gsem[parity], wsem[parity])
            si, ro, ov, gs, ws = pp
            wait(si, ro, gs)
            pltpu.make_async_copy(
                ov, out_hbm.at[h, :, pl.ds(bc0, _NBC)], ws).wait()
            transpose(ro, ov)
            pltpu.async_copy(ov, out_hbm.at[h, :, pl.ds(bc0, _NBC)], ws)

            @pl.when(h + 2 < _HIST)
            def _():
                extract(h + 2, si)
                start(si, ro, gs)

        half(h0, 0)
        half(h0 + 1, 1)
        return carry

    lax.fori_loop(1, _HIST // 2, group, 0)
    pltpu.make_async_copy(
        outv0, out_hbm.at[_HIST - 2, :, pl.ds(bc0, _NBC)], wsem0).wait()
    pltpu.make_async_copy(
        outv1, out_hbm.at[_HIST - 1, :, pl.ds(bc0, _NBC)], wsem1).wait()


@jax.jit
def kernel(indices, table):
    flat_idx = indices.reshape(-1).astype(jnp.int32)
    mesh = plsc.VectorSubcoreMesh(core_axis_name="c", subcore_axis_name="s")
    scratch = [
        pltpu.VMEM((_BPW * _HIST,), jnp.int32),
        pltpu.VMEM((_BPW,), jnp.int32),
        pltpu.VMEM((_BPW,), jnp.int32),
        pltpu.VMEM((_BPW, _D), jnp.float32),
        pltpu.VMEM((_BPW, _D), jnp.float32),
        pltpu.VMEM((2, _NBC, 8, 128), jnp.float32),
        pltpu.VMEM((2, _NBC, 8, 128), jnp.float32),
        pltpu.SemaphoreType.DMA,
        pltpu.SemaphoreType.DMA,
        pltpu.SemaphoreType.DMA,
        pltpu.SemaphoreType.DMA,
    ]
    out5 = pl.kernel(
        _body,
        out_type=jax.ShapeDtypeStruct((_HIST, 2, 128, 8, 128), jnp.float32),
        mesh=mesh,
        scratch_types=scratch,
        compiler_params=pltpu.CompilerParams(
            use_tc_tiling_on_sc=False, needs_layout_passes=False),
    )(flat_idx, table)
    return out5.transpose(2, 4, 0, 1, 3).reshape(_BATCH, _HIST, _D)


# speedup vs baseline: 1.0587x; 1.0587x over previous
import jax
import jax.numpy as jnp
from jax import lax
from jax.experimental import pallas as pl
from jax.experimental.pallas import tpu as pltpu
from jax.experimental.pallas import tpu_sc as plsc

_NC = 2
_NS = 16
_NW = _NC * _NS
_BATCH = 16384
_HIST = 50
_D = 16
_BPW = _BATCH // _NW
_NBC = _BPW // 128


def _body(idx_hbm, table_hbm, out_hbm, idx_v, sidx0, sidx1, rows0, rows1,
          outv0, outv1, gsem0, gsem1, wsem0, wsem1):
    wid = lax.axis_index("s") * _NC + lax.axis_index("c")
    b0 = wid * _BPW
    bc0 = wid * _NBC
    pltpu.sync_copy(idx_hbm.at[pl.ds(b0 * _HIST, _BPW * _HIST)], idx_v)
    lane = lax.iota(jnp.int32, 16)

    def extract(h, sidx):
        for k in range(_BPW // 16):
            pos = lane * _HIST + (k * 16 * _HIST + h)
            sidx[pl.ds(k * 16, 16)] = plsc.load_gather(idx_v, [pos])

    def start(sidx, rows, sem):
        return pltpu.async_copy(table_hbm.at[sidx], rows, sem)

    def wait(sidx, rows, sem):
        pltpu.make_async_copy(table_hbm.at[sidx], rows, sem).wait()

    def transpose(rows, outv):
        for tr in range(2):
            for bcl in range(_NBC):
                for r in range(8):
                    e = jnp.full((16,), tr * 8 + r, jnp.int32)
                    for bk in range(8):
                        ridx = lane + (bcl * 128 + bk * 16)
                        outv[tr, bcl, r, pl.ds(bk * 16, 16)] = (
                            plsc.load_gather(rows, [ridx, e]))

    def start_write(h, outv, wsem):
        return pltpu.async_copy(
            outv, out_hbm.at[h, :, pl.ds(bc0, _NBC)], wsem)

    def wait_write(h, outv, wsem):
        pltpu.make_async_copy(
            outv, out_hbm.at[h, :, pl.ds(bc0, _NBC)], wsem).wait()

    def drain(h, ro, ov):
        transpose(ro, ov)
        pltpu.sync_copy(ov.at[0], out_hbm.at[h, 0, pl.ds(bc0, _NBC)])
        pltpu.sync_copy(ov.at[1], out_hbm.at[h, 1, pl.ds(bc0, _NBC)])

    extract(0, sidx0)
    start(sidx0, rows0, gsem0)

    def group(g, carry):
        h0 = g * 2
        extract(h0 + 1, sidx1)
        start(sidx1, rows1, gsem1)
        wait(sidx0, rows0, gsem0)
        drain(h0, rows0, outv0)

        @pl.when(h0 + 2 < _HIST)
        def _():
            extract(h0 + 2, sidx0)
            start(sidx0, rows0, gsem0)

        wait(sidx1, rows1, gsem1)
        drain(h0 + 1, rows1, outv1)
        return carry

    lax.fori_loop(0, _HIST // 2, group, 0)


@jax.jit
def kernel(indices, table):
    flat_idx = indices.reshape(-1).astype(jnp.int32)
    mesh = plsc.VectorSubcoreMesh(core_axis_name="c", subcore_axis_name="s")
    scratch = [
        pltpu.VMEM((_BPW * _HIST,), jnp.int32),
        pltpu.VMEM((_BPW,), jnp.int32),
        pltpu.VMEM((_BPW,), jnp.int32),
        pltpu.VMEM((_BPW, _D), jnp.float32),
        pltpu.VMEM((_BPW, _D), jnp.float32),
        pltpu.VMEM((2, _NBC, 8, 128), jnp.float32),
        pltpu.VMEM((2, _NBC, 8, 128), jnp.float32),
        pltpu.SemaphoreType.DMA,
        pltpu.SemaphoreType.DMA,
        pltpu.SemaphoreType.DMA,
        pltpu.SemaphoreType.DMA,
    ]
    out5 = pl.kernel(
        _body,
        out_type=jax.ShapeDtypeStruct((_HIST, 2, 128, 8, 128), jnp.float32),
        mesh=mesh,
        scratch_types=scratch,
        compiler_params=pltpu.CompilerParams(
            use_tc_tiling_on_sc=False, needs_layout_passes=False),
    )(flat_idx, table)
    return out5.transpose(2, 4, 0, 1, 3).reshape(_BATCH, _HIST, _D)


# speedup vs baseline: 1.2701x; 1.1997x over previous
import jax
import jax.numpy as jnp
from jax import lax
from jax.experimental import pallas as pl
from jax.experimental.pallas import tpu as pltpu
from jax.experimental.pallas import tpu_sc as plsc

_NC = 2
_NS = 16
_NW = _NC * _NS
_BATCH = 16384
_HIST = 50
_D = 16
_BPW = _BATCH // _NW
_NBC = _BPW // 128


def _body(idx_hbm, table_hbm, out_hbm, idx_v, sidx0, sidx1, rows0, rows1,
          outv0, outv1, gsem0, gsem1, wsem0, wsem1):
    wid = lax.axis_index("s") * _NC + lax.axis_index("c")
    b0 = wid * _BPW
    bc0 = wid * _NBC
    pltpu.sync_copy(idx_hbm.at[pl.ds(b0 * _HIST, _BPW * _HIST)], idx_v)
    lane = lax.iota(jnp.int32, 16)

    def extract(h, sidx):
        for k in range(_BPW // 16):
            pos = lane * _HIST + (k * 16 * _HIST + h)
            sidx[pl.ds(k * 16, 16)] = plsc.load_gather(idx_v, [pos])

    def start(sidx, rows, sem):
        return pltpu.async_copy(table_hbm.at[sidx], rows, sem)

    def wait(sidx, rows, sem):
        pltpu.make_async_copy(table_hbm.at[sidx], rows, sem).wait()

    def transpose(rows, outv):
        for tr in range(2):
            for bcl in range(_NBC):
                for r in range(8):
                    e = jnp.full((16,), tr * 8 + r, jnp.int32)
                    for bk in range(8):
                        ridx = lane + (bcl * 128 + bk * 16)
                        outv[tr, bcl, r, pl.ds(bk * 16, 16)] = (
                            plsc.load_gather(rows, [ridx, e]))

    def start_write(h, outv, wsem):
        return pltpu.async_copy(
            outv, out_hbm.at[h, :, pl.ds(bc0, _NBC)], wsem)

    def wait_write(h, outv, wsem):
        pltpu.make_async_copy(
            outv, out_hbm.at[h, :, pl.ds(bc0, _NBC)], wsem).wait()

    def drain(h, ro, ov):
        transpose(ro, ov)
        pltpu.sync_copy(ov.at[0], out_hbm.at[h, 0, pl.ds(bc0, _NBC)])
        pltpu.sync_copy(ov.at[1], out_hbm.at[h, 1, pl.ds(bc0, _NBC)])

    extract(0, sidx0)
    start(sidx0, rows0, gsem0)

    def group(g, carry):
        h0 = g * 2
        extract(h0 + 1, sidx1)
        start(sidx1, rows1, gsem1)
        wait(sidx0, rows0, gsem0)
        drain(h0, rows0, outv0)

        @pl.when(h0 + 2 < _HIST)
        def _():
            extract(h0 + 2, sidx0)
            start(sidx0, rows0, gsem0)

        wait(sidx1, rows1, gsem1)
        drain(h0 + 1, rows1, outv1)
        return carry

    lax.fori_loop(0, _HIST // 2, group, 0)


_TBC = 8192


def _detile_body(tt_ref, o_ref):
    bt = tt_ref[...].T.reshape(_TBC // 8, 8, _D)
    o_ref[...] = jnp.concatenate([bt[:, s, :] for s in range(8)], axis=1)


def _detile_table(table):
    t2 = pl.pallas_call(
        _detile_body,
        grid=((1000000 + _TBC - 1) // _TBC,),
        in_specs=[pl.BlockSpec((16, _TBC), lambda j: (0, j))],
        out_specs=pl.BlockSpec((_TBC // 8, 128), lambda j: (j, 0)),
        out_shape=jax.ShapeDtypeStruct((1000000 * 16 // 128, 128),
                                       jnp.float32),
    )(table.T)
    return t2.reshape(1000000, _D)


@jax.jit
def kernel(indices, table):
    flat_idx = indices.reshape(-1).astype(jnp.int32)
    table_lin = _detile_table(table)
    mesh = plsc.VectorSubcoreMesh(core_axis_name="c", subcore_axis_name="s")
    scratch = [
        pltpu.VMEM((_BPW * _HIST,), jnp.int32),
        pltpu.VMEM((_BPW,), jnp.int32),
        pltpu.VMEM((_BPW,), jnp.int32),
        pltpu.VMEM((_BPW, _D), jnp.float32),
        pltpu.VMEM((_BPW, _D), jnp.float32),
        pltpu.VMEM((2, _NBC, 8, 128), jnp.float32),
        pltpu.VMEM((2, _NBC, 8, 128), jnp.float32),
        pltpu.SemaphoreType.DMA,
        pltpu.SemaphoreType.DMA,
        pltpu.SemaphoreType.DMA,
        pltpu.SemaphoreType.DMA,
    ]
    out5 = pl.kernel(
        _body,
        out_type=jax.ShapeDtypeStruct((_HIST, 2, 128, 8, 128), jnp.float32),
        mesh=mesh,
        scratch_types=scratch,
        compiler_params=pltpu.CompilerParams(
            use_tc_tiling_on_sc=False, needs_layout_passes=False),
    )(flat_idx, table_lin)
    return out5.transpose(2, 4, 0, 1, 3).reshape(_BATCH, _HIST, _D)


# speedup vs baseline: 1.3679x; 1.0770x over previous
import jax
import jax.numpy as jnp
from jax import lax
from jax.experimental import pallas as pl
from jax.experimental.pallas import tpu as pltpu
from jax.experimental.pallas import tpu_sc as plsc

_NC = 2
_NS = 16
_NW = _NC * _NS
_BATCH = 16384
_HIST = 50
_D = 16
_BPW = _BATCH // _NW
_NBC = _BPW // 128


def _body(idx_hbm, table_hbm, out_hbm, idx_v, sidx0, sidx1, rows0, rows1,
          outv0, outv1, gsem0, gsem1, wsem0, wsem1):
    wid = lax.axis_index("s") * _NC + lax.axis_index("c")
    b0 = wid * _BPW
    bc0 = wid * _NBC
    pltpu.sync_copy(idx_hbm.at[pl.ds(b0 * _HIST, _BPW * _HIST)], idx_v)
    lane = lax.iota(jnp.int32, 16)

    def extract(h, sidx):
        for k in range(_BPW // 16):
            pos = lane * _HIST + (k * 16 * _HIST + h)
            sidx[pl.ds(k * 16, 16)] = plsc.load_gather(idx_v, [pos])

    def start(sidx, rows, sem):
        return pltpu.async_copy(table_hbm.at[sidx], rows, sem)

    def wait(sidx, rows, sem):
        pltpu.make_async_copy(table_hbm.at[sidx], rows, sem).wait()

    def transpose(rows, outv):
        for tr in range(2):
            for bcl in range(_NBC):
                for r in range(8):
                    e = jnp.full((16,), tr * 8 + r, jnp.int32)
                    for bk in range(8):
                        ridx = lane + (bcl * 128 + bk * 16)
                        outv[tr, bcl, r, pl.ds(bk * 16, 16)] = (
                            plsc.load_gather(rows, [ridx, e]))

    def start_write(h, outv, wsem):
        return pltpu.async_copy(
            outv, out_hbm.at[h, :, pl.ds(bc0, _NBC)], wsem)

    def wait_write(h, outv, wsem):
        pltpu.make_async_copy(
            outv, out_hbm.at[h, :, pl.ds(bc0, _NBC)], wsem).wait()

    def drain(h, ro, ov):
        transpose(ro, ov)
        pltpu.sync_copy(ov.at[0], out_hbm.at[h, 0, pl.ds(bc0, _NBC)])
        pltpu.sync_copy(ov.at[1], out_hbm.at[h, 1, pl.ds(bc0, _NBC)])

    extract(0, sidx0)
    start(sidx0, rows0, gsem0)

    def group(g, carry):
        h0 = g * 2
        extract(h0 + 1, sidx1)
        start(sidx1, rows1, gsem1)
        wait(sidx0, rows0, gsem0)
        drain(h0, rows0, outv0)

        @pl.when(h0 + 2 < _HIST)
        def _():
            extract(h0 + 2, sidx0)
            start(sidx0, rows0, gsem0)

        wait(sidx1, rows1, gsem1)
        drain(h0 + 1, rows1, outv1)
        return carry

    lax.fori_loop(0, _HIST // 2, group, 0)


_TBC = 8192


def _detile_body(tt_ref, o_ref):
    bt = tt_ref[...].T.reshape(_TBC // 8, 8, _D)
    for s in range(8):
        o_ref[:, s * _D:(s + 1) * _D] = bt[:, s, :]


def _detile_table(table):
    t2 = pl.pallas_call(
        _detile_body,
        grid=((1000000 + _TBC - 1) // _TBC,),
        in_specs=[pl.BlockSpec((16, _TBC), lambda j: (0, j))],
        out_specs=pl.BlockSpec((_TBC // 8, 128), lambda j: (j, 0)),
        out_shape=jax.ShapeDtypeStruct((1000000 * 16 // 128, 128),
                                       jnp.float32),
    )(table.T)
    return t2.reshape(1000000, _D)


@jax.jit
def kernel(indices, table):
    flat_idx = indices.reshape(-1).astype(jnp.int32)
    table_lin = _detile_table(table)
    mesh = plsc.VectorSubcoreMesh(core_axis_name="c", subcore_axis_name="s")
    scratch = [
        pltpu.VMEM((_BPW * _HIST,), jnp.int32),
        pltpu.VMEM((_BPW,), jnp.int32),
        pltpu.VMEM((_BPW,), jnp.int32),
        pltpu.VMEM((_BPW, _D), jnp.float32),
        pltpu.VMEM((_BPW, _D), jnp.float32),
        pltpu.VMEM((2, _NBC, 8, 128), jnp.float32),
        pltpu.VMEM((2, _NBC, 8, 128), jnp.float32),
        pltpu.SemaphoreType.DMA,
        pltpu.SemaphoreType.DMA,
        pltpu.SemaphoreType.DMA,
        pltpu.SemaphoreType.DMA,
    ]
    out5 = pl.kernel(
        _body,
        out_type=jax.ShapeDtypeStruct((_HIST, 2, 128, 8, 128), jnp.float32),
        mesh=mesh,
        scratch_types=scratch,
        compiler_params=pltpu.CompilerParams(
            use_tc_tiling_on_sc=False, needs_layout_passes=False),
    )(flat_idx, table_lin)
    return out5.transpose(2, 4, 0, 1, 3).reshape(_BATCH, _HIST, _D)


# speedup vs baseline: 1.3805x; 1.0092x over previous
import jax
import jax.numpy as jnp
from jax import lax
from jax.experimental import pallas as pl
from jax.experimental.pallas import tpu as pltpu
from jax.experimental.pallas import tpu_sc as plsc

_NC = 2
_NS = 16
_NW = _NC * _NS
_BATCH = 16384
_HIST = 50
_D = 16
_BPW = _BATCH // _NW
_NBC = _BPW // 128


def _body(idx_hbm, table_hbm, out_hbm, idx_v, sidx0, sidx1, rows0, rows1,
          outv0, outv1, gsem0, gsem1, wsem0, wsem1):
    wid = lax.axis_index("s") * _NC + lax.axis_index("c")
    b0 = wid * _BPW
    bc0 = wid * _NBC
    pltpu.sync_copy(idx_hbm.at[pl.ds(b0 * _HIST, _BPW * _HIST)], idx_v)
    lane = lax.iota(jnp.int32, 16)

    def extract(h, sidx):
        for k in range(_BPW // 16):
            pos = lane * _HIST + (k * 16 * _HIST + h)
            sidx[pl.ds(k * 16, 16)] = plsc.load_gather(idx_v, [pos])

    def start(sidx, rows, sem):
        return pltpu.async_copy(table_hbm.at[sidx], rows, sem)

    def wait(sidx, rows, sem):
        pltpu.make_async_copy(table_hbm.at[sidx], rows, sem).wait()

    def transpose(rows, outv):
        for tr in range(2):
            for bcl in range(_NBC):
                for r in range(8):
                    e = jnp.full((16,), tr * 8 + r, jnp.int32)
                    for bk in range(8):
                        ridx = lane + (bcl * 128 + bk * 16)
                        outv[tr, bcl, r, pl.ds(bk * 16, 16)] = (
                            plsc.load_gather(rows, [ridx, e]))

    def start_write(h, outv, wsem):
        return pltpu.async_copy(
            outv, out_hbm.at[h, :, pl.ds(bc0, _NBC)], wsem)

    def wait_write(h, outv, wsem):
        pltpu.make_async_copy(
            outv, out_hbm.at[h, :, pl.ds(bc0, _NBC)], wsem).wait()

    def drain(h, ro, ov):
        transpose(ro, ov)
        pltpu.sync_copy(ov.at[0], out_hbm.at[h, 0, pl.ds(bc0, _NBC)])
        pltpu.sync_copy(ov.at[1], out_hbm.at[h, 1, pl.ds(bc0, _NBC)])

    extract(0, sidx0)
    start(sidx0, rows0, gsem0)

    def group(g, carry):
        h0 = g * 2
        extract(h0 + 1, sidx1)
        start(sidx1, rows1, gsem1)
        wait(sidx0, rows0, gsem0)
        drain(h0, rows0, outv0)

        @pl.when(h0 + 2 < _HIST)
        def _():
            extract(h0 + 2, sidx0)
            start(sidx0, rows0, gsem0)

        wait(sidx1, rows1, gsem1)
        drain(h0 + 1, rows1, outv1)
        return carry

    lax.fori_loop(0, _HIST // 2, group, 0)


_TBC = 16384


def _detile_body(tt_ref, o_ref):
    bt = tt_ref[...].T.reshape(_TBC // 8, 8, _D)
    for s in range(8):
        o_ref[:, s * _D:(s + 1) * _D] = bt[:, s, :]


def _detile_table(table):
    t2 = pl.pallas_call(
        _detile_body,
        grid=((1000000 + _TBC - 1) // _TBC,),
        in_specs=[pl.BlockSpec((16, _TBC), lambda j: (0, j))],
        out_specs=pl.BlockSpec((_TBC // 8, 128), lambda j: (j, 0)),
        out_shape=jax.ShapeDtypeStruct((1000000 * 16 // 128, 128),
                                       jnp.float32),
    )(table.T)
    return t2.reshape(1000000, _D)


@jax.jit
def kernel(indices, table):
    flat_idx = indices.reshape(-1).astype(jnp.int32)
    table_lin = _detile_table(table)
    mesh = plsc.VectorSubcoreMesh(core_axis_name="c", subcore_axis_name="s")
    scratch = [
        pltpu.VMEM((_BPW * _HIST,), jnp.int32),
        pltpu.VMEM((_BPW,), jnp.int32),
        pltpu.VMEM((_BPW,), jnp.int32),
        pltpu.VMEM((_BPW, _D), jnp.float32),
        pltpu.VMEM((_BPW, _D), jnp.float32),
        pltpu.VMEM((2, _NBC, 8, 128), jnp.float32),
        pltpu.VMEM((2, _NBC, 8, 128), jnp.float32),
        pltpu.SemaphoreType.DMA,
        pltpu.SemaphoreType.DMA,
        pltpu.SemaphoreType.DMA,
        pltpu.SemaphoreType.DMA,
    ]
    out5 = pl.kernel(
        _body,
        out_type=jax.ShapeDtypeStruct((_HIST, 2, 128, 8, 128), jnp.float32),
        mesh=mesh,
        scratch_types=scratch,
        compiler_params=pltpu.CompilerParams(
            use_tc_tiling_on_sc=False, needs_layout_passes=False),
    )(flat_idx, table_lin)
    return out5.transpose(2, 4, 0, 1, 3).reshape(_BATCH, _HIST, _D)
